# Initial kernel scaffold; baseline (speedup 1.0000x reference)
import jax
import jax.numpy as jnp
from jax.experimental import pallas as pl


def kernel(logits, labels):
    raise NotImplementedError("write your pallas kernel here")


# speedup vs baseline: 1.6820x; 1.6820x over previous
import functools

import jax
import jax.numpy as jnp
from jax.experimental import pallas as pl

_SCALE = 64.0
_MARGIN = 0.5
_ROWS = 8


def _arcface_block(x_ref, lab_ref, o_ref):
    x = x_ref[...]
    lab = lab_ref[...]
    valid = lab != -1
    lab_safe = jnp.where(valid, lab, 0)

    inv = jax.lax.rsqrt(jnp.maximum(jnp.sum(x * x, axis=1, keepdims=True),
                                    1e-24))

    cols = jax.lax.broadcasted_iota(jnp.int32, x.shape, 1)
    mask = cols == lab_safe
    t = jnp.sum(jnp.where(mask, x, 0.0), axis=1, keepdims=True) * inv
    t_clip = jnp.clip(t, -1.0, 1.0)
    with_margin = (t_clip * jnp.float32(jnp.cos(_MARGIN))
                   - jnp.float32(jnp.sin(_MARGIN))
                   * jnp.sqrt(jnp.maximum(1.0 - t_clip * t_clip, 0.0)))
    new_val = jnp.where(valid, with_margin, t)

    o_ref[...] = jnp.where(mask, new_val * _SCALE, x * (inv * _SCALE))


@jax.jit
def _run(logits, labels2d):
    n, c = logits.shape
    return pl.pallas_call(
        _arcface_block,
        grid=(n // _ROWS,),
        in_specs=[
            pl.BlockSpec((_ROWS, c), lambda i: (i, 0)),
            pl.BlockSpec((_ROWS, 1), lambda i: (i, 0)),
        ],
        out_specs=pl.BlockSpec((_ROWS, c), lambda i: (i, 0)),
        out_shape=jax.ShapeDtypeStruct((n, c), jnp.float32),
    )(logits, labels2d)


def kernel(logits, labels):
    labels2d = labels.astype(jnp.int32).reshape(-1, 1)
    return _run(logits, labels2d)


# speedup vs baseline: 1.7762x; 1.0560x over previous
import functools

import jax
import jax.numpy as jnp
from jax.experimental import pallas as pl
from jax.experimental.pallas import tpu as pltpu

_SCALE = 64.0
_MARGIN = 0.5
_ROWS = 16


def _arcface_block(x_ref, lab_ref, o_ref):
    x = x_ref[...]
    lab = lab_ref[...]
    valid = lab != -1
    lab_safe = jnp.where(valid, lab, 0)

    inv = jax.lax.rsqrt(jnp.maximum(jnp.sum(x * x, axis=1, keepdims=True),
                                    1e-24))

    cols = jax.lax.broadcasted_iota(jnp.int32, x.shape, 1)
    mask = cols == lab_safe
    t = jnp.sum(jnp.where(mask, x, 0.0), axis=1, keepdims=True) * inv
    t_clip = jnp.clip(t, -1.0, 1.0)
    with_margin = (t_clip * jnp.float32(jnp.cos(_MARGIN))
                   - jnp.float32(jnp.sin(_MARGIN))
                   * jnp.sqrt(jnp.maximum(1.0 - t_clip * t_clip, 0.0)))
    new_val = jnp.where(valid, with_margin, t)

    o_ref[...] = jnp.where(mask, new_val * _SCALE, x * (inv * _SCALE))


@jax.jit
def _run(logits, labels2d):
    n, c = logits.shape
    return pl.pallas_call(
        _arcface_block,
        grid=(n // _ROWS,),
        in_specs=[
            pl.BlockSpec((_ROWS, c), lambda i: (i, 0)),
            pl.BlockSpec((_ROWS, 1), lambda i: (i, 0)),
        ],
        out_specs=pl.BlockSpec((_ROWS, c), lambda i: (i, 0)),
        out_shape=jax.ShapeDtypeStruct((n, c), jnp.float32),
        compiler_params=pltpu.CompilerParams(
            dimension_semantics=("parallel",),
        ),
    )(logits, labels2d)


def kernel(logits, labels):
    labels2d = labels.astype(jnp.int32).reshape(-1, 1)
    return _run(logits, labels2d)
